# Initial kernel scaffold; baseline (speedup 1.0000x reference)
import jax
import jax.numpy as jnp
from jax.experimental import pallas as pl


def kernel(text_emb, brand_id, color_id, price_oneh, brand_table, color_table, W_price, W1, b1, W2, b2):
    raise NotImplementedError("write your pallas kernel here")


# speedup vs baseline: 1.1742x; 1.1742x over previous
import functools

import jax
import jax.numpy as jnp
from jax import lax
from jax.experimental import pallas as pl
from jax.experimental.pallas import tpu as pltpu
from jax.experimental.pallas import tpu_sc as plsc

B = 16384
NC, NS = 2, 16
NW = NC * NS
B_PER_W = B // NW

BLK = 2048


def _sc_gather_both(brand_tab128, bidx, color_tab128, cidx):
    mesh = plsc.VectorSubcoreMesh(core_axis_name="c", subcore_axis_name="s")

    @functools.partial(
        pl.kernel,
        mesh=mesh,
        out_type=(
            jax.ShapeDtypeStruct((B, 128), jnp.float32),
            jax.ShapeDtypeStruct((B, 128), jnp.float32),
        ),
        scratch_types=[
            pltpu.VMEM((B_PER_W,), jnp.int32),
            pltpu.VMEM((B_PER_W, 128), jnp.float32),
            pltpu.SemaphoreType.DMA,
        ],
    )
    def k(btab_hbm, bidx_hbm, ctab_hbm, cidx_hbm, be_hbm, ce_hbm,
          idx_v, rows_v, sem):
        wid = lax.axis_index("s") * NC + lax.axis_index("c")
        base = wid * B_PER_W
        pltpu.sync_copy(bidx_hbm.at[pl.ds(base, B_PER_W)], idx_v)
        pltpu.async_copy(btab_hbm.at[idx_v], rows_v, sem).wait()
        pltpu.sync_copy(rows_v, be_hbm.at[pl.ds(base, B_PER_W)])
        pltpu.sync_copy(cidx_hbm.at[pl.ds(base, B_PER_W)], idx_v)
        pltpu.async_copy(ctab_hbm.at[idx_v], rows_v, sem).wait()
        pltpu.sync_copy(rows_v, ce_hbm.at[pl.ds(base, B_PER_W)])

    return k(brand_tab128, bidx, color_tab128, cidx)


def _tc_tower(text_ref, be128_ref, ce128_ref, bid_ref, cid_ref, price_ref,
              w1t_ref, w1b_ref, w1c_ref, w1p_ref, wp_ref,
              b1_ref, w2_ref, b2_ref, out_ref):
    f32 = jnp.float32
    bsel = lax.rem(bid_ref[...], 4)
    be128 = be128_ref[...]
    be = jnp.zeros((be128.shape[0], 32), f32)
    for j in range(4):
        be += jnp.where(bsel == j, be128[:, 32 * j:32 * (j + 1)], 0.0)
    csel = lax.rem(cid_ref[...], 8)
    ce128 = ce128_ref[...]
    ce = jnp.zeros((ce128.shape[0], 16), f32)
    for j in range(8):
        ce += jnp.where(csel == j, ce128[:, 16 * j:16 * (j + 1)], 0.0)

    pe = jnp.dot(price_ref[...], wp_ref[...], preferred_element_type=f32)
    h = jnp.dot(text_ref[...], w1t_ref[...], preferred_element_type=f32)
    h += jnp.dot(be, w1b_ref[...], preferred_element_type=f32)
    h += jnp.dot(ce, w1c_ref[...], preferred_element_type=f32)
    h += jnp.dot(pe, w1p_ref[...], preferred_element_type=f32)
    h = jnp.maximum(h + b1_ref[...], 0.0)
    z = jnp.dot(h, w2_ref[...], preferred_element_type=f32) + b2_ref[...]
    norm = jnp.sqrt(jnp.sum(z * z, axis=1, keepdims=True))
    out_ref[...] = z * (1.0 / jnp.maximum(norm, 1e-12))


def kernel(text_emb, brand_id, color_id, price_oneh, brand_table, color_table,
           W_price, W1, b1, W2, b2):
    btab128 = brand_table.reshape(-1, 128)
    ctab128 = color_table.reshape(-1, 128)
    bidx = lax.div(brand_id, 4)
    cidx = lax.div(color_id, 8)

    be128, ce128 = _sc_gather_both(btab128, bidx, ctab128, cidx)

    w1t = W1[:, :384].T
    w1b = W1[:, 384:416].T
    w1c = W1[:, 416:432].T
    w1p = W1[:, 432:448].T
    wp = W_price.T
    w2 = W2.T
    b1r = b1.reshape(1, 256)
    b2r = b2.reshape(1, 128)
    bid2 = brand_id.reshape(B, 1)
    cid2 = color_id.reshape(B, 1)

    grid = (B // BLK,)
    row_spec = lambda w: pl.BlockSpec((BLK, w), lambda i: (i, 0))
    full_spec = lambda a, b: pl.BlockSpec((a, b), lambda i: (0, 0))

    out = pl.pallas_call(
        _tc_tower,
        grid=grid,
        in_specs=[
            row_spec(384),
            row_spec(128),
            row_spec(128),
            row_spec(1),
            row_spec(1),
            row_spec(100),
            full_spec(384, 256),
            full_spec(32, 256),
            full_spec(16, 256),
            full_spec(16, 256),
            full_spec(100, 16),
            full_spec(1, 256),
            full_spec(256, 128),
            full_spec(1, 128),
        ],
        out_specs=row_spec(128),
        out_shape=jax.ShapeDtypeStruct((B, 128), jnp.float32),
    )(text_emb, be128, ce128, bid2, cid2, price_oneh,
      w1t, w1b, w1c, w1p, wp, b1r, w2, b2r)
    return out
